# baseline (speedup 1.0000x reference)
import functools

import jax
import jax.numpy as jnp
from jax.experimental import pallas as pl
from jax.experimental.pallas import tpu as pltpu

_RATE = 0.5
_ENC = [(3, 8, 5, 1, 2), (8, 16, 3, 2, 1), (16, 32, 3, 2, 1), (32, 64, 3, 2, 1), (64, 64, 3, 2, 1)]


def _norm_act_body(x_ref, g_ref, be_ref, o_ref, *, cnt, act):
    c = pl.program_id(0)
    xb = x_ref[:, 0]
    m = jnp.sum(xb) / cnt
    xc = xb - m
    v = jnp.sum(xc * xc) / cnt
    y = xc / jnp.sqrt(v + 1e-5) * g_ref[c] + be_ref[c]
    if act == 'lrelu':
        y = jnp.where(y >= 0, y, 0.2 * y)
    o_ref[:, 0] = y


def _norm_act(x, g, be, act='lrelu'):
    n, c, h, w = x.shape
    body = functools.partial(_norm_act_body, cnt=float(n * h * w), act=act)
    return pl.pallas_call(
        body,
        grid=(c,),
        in_specs=[
            pl.BlockSpec((n, 1, h, w), lambda i: (0, i, 0, 0)),
            pl.BlockSpec(memory_space=pltpu.SMEM),
            pl.BlockSpec(memory_space=pltpu.SMEM),
        ],
        out_specs=pl.BlockSpec((n, 1, h, w), lambda i: (0, i, 0, 0)),
        out_shape=jax.ShapeDtypeStruct((n, c, h, w), jnp.float32),
    )(x, g, be)


def _conv_block(x, w, b, g, be, stride, pad, norm=True, act='lrelu'):
    if pad > 0:
        x = jnp.pad(x, ((0, 0), (0, 0), (pad, pad), (pad, pad)), mode='reflect')
    x = jax.lax.conv_general_dilated(x, w, (stride, stride), 'VALID',
                                     dimension_numbers=('NCHW', 'OIHW', 'NCHW'))
    if norm:
        return _norm_act(x, g, be, act)
    x = x + b[None, :, None, None]
    if act == 'lrelu':
        x = jnp.where(x >= 0, x, 0.2 * x)
    elif act == 'tanh':
        x = jnp.tanh(x)
    return x


def _up2(x):
    return jnp.repeat(jnp.repeat(x, 2, axis=2), 2, axis=3)


def _fusion_body(q_ref, fi_ref, sim_ref, out_ref, *, n, hw, num):
    b_idx = pl.program_id(0)
    featf = q_ref[0, 0]
    fi = fi_ref[0]

    fnorm = jnp.sqrt(jnp.sum(featf * featf, axis=0, keepdims=True))
    wf = featf / jnp.maximum(fnorm, 1e-12)

    hidx = jax.lax.broadcasted_iota(jnp.int32, (hw, num), 0)
    G = (hidx == fi).astype(jnp.float32)

    feat_sel = jax.lax.dot(featf, G)
    wfs = jax.lax.dot(wf, G)
    wnorm = jnp.sqrt(jnp.sum(wfs * wfs, axis=0, keepdims=True))
    wfs = wfs / jnp.maximum(wnorm, 1e-12)

    acc = sim_ref[b_idx, 0] * feat_sel
    miota = jax.lax.broadcasted_iota(jnp.int32, (num, hw), 1)
    for j in range(n):
        ref_j = q_ref[0, 1 + j]
        rnorm = jnp.sqrt(jnp.sum(ref_j * ref_j, axis=0, keepdims=True))
        wr = ref_j / jnp.maximum(rnorm, 1e-12)
        fx = jax.lax.dot_general(wfs, wr, (((0,), (0,)), ((), ())))
        maxv = jnp.max(fx, axis=1, keepdims=True)
        eligible = fx >= maxv
        ind = jnp.min(jnp.where(eligible, miota, hw), axis=1, keepdims=True)
        onehot = (miota == ind).astype(jnp.float32)
        ref_sel = jax.lax.dot_general(ref_j, onehot, (((1,), (1,)), ((), ())))
        acc = acc + sim_ref[b_idx, 1 + j] * ref_sel

    covered = jax.lax.dot_general(jnp.ones((1, num), jnp.float32), G,
                                  (((1,), (1,)), ((), ())))
    scattered = jax.lax.dot_general(acc, G, (((1,), (1,)), ((), ())))
    out_ref[0] = featf * (1.0 - covered) + scattered


def _fusion_pallas(q, fi, sim):
    b, k, c, hw = q.shape
    n = k - 1
    num = fi.shape[2]
    body = functools.partial(_fusion_body, n=n, hw=hw, num=num)
    return pl.pallas_call(
        body,
        grid=(b,),
        in_specs=[
            pl.BlockSpec((1, k, c, hw), lambda i: (i, 0, 0, 0)),
            pl.BlockSpec((1, 1, num), lambda i: (i, 0, 0)),
            pl.BlockSpec(memory_space=pltpu.SMEM),
        ],
        out_specs=pl.BlockSpec((1, c, hw), lambda i: (i, 0, 0)),
        out_shape=jax.ShapeDtypeStruct((b, c, hw), jnp.float32),
    )(q, fi, sim)


def kernel(xs, params):
    b, k, cc, hh, ww = xs.shape
    x = xs.reshape(b * k, cc, hh, ww)
    feats = []
    for li, (ci, co, kk, st, pd) in enumerate(_ENC):
        x = _conv_block(x, params['enc%d_w' % li], params['enc%d_b' % li],
                        params['enc%d_g' % li], params['enc%d_be' % li], st, pd)
        feats.append(x)
    x5 = feats[-1]
    c, h, w = x5.shape[1], x5.shape[2], x5.shape[3]
    hw = h * w
    num = int(_RATE * hw)
    sim = jax.random.uniform(jax.random.key(42), (b, k), jnp.float32)
    sim = sim / jnp.sum(sim, axis=1, keepdims=True)

    idx_keys = jax.random.split(jax.random.key(7), b)
    feat_indices = jnp.stack(
        [jax.random.permutation(idx_keys[i], hw)[:num] for i in range(b)])
    fi = feat_indices.astype(jnp.int32).reshape(b, 1, num)

    q = x5.reshape(b, k, c, hw)
    feat_gen = _fusion_pallas(q, fi, sim).reshape(b, c, h, w)

    skips = [f.reshape(b, k, f.shape[1], f.shape[2], f.shape[3])[:, 0]
             for f in feats[:-1]]
    x = _up2(feat_gen)
    s4 = _conv_block(skips[3], params['skip1_w'], params['skip1_b'],
                     params['skip1_g'], params['skip1_be'], 1, 0)
    x = jnp.concatenate([x, s4], axis=1)
    x = _conv_block(x, params['conv1_w'], params['conv1_b'],
                    params['conv1_g'], params['conv1_be'], 1, 1)
    x = _up2(x)
    s3 = _conv_block(skips[2], params['skip2_w'], params['skip2_b'],
                     params['skip2_g'], params['skip2_be'], 1, 0)
    x = jnp.concatenate([x, s3], axis=1)
    x = _conv_block(x, params['conv2_w'], params['conv2_b'],
                    params['conv2_g'], params['conv2_be'], 1, 1)
    x = _up2(x)
    x = _conv_block(x, params['conv3_w'], params['conv3_b'],
                    params['conv3_g'], params['conv3_be'], 1, 1)
    x = _up2(x)
    x = _conv_block(x, params['conv4_w'], params['conv4_b'],
                    params['conv4_g'], params['conv4_be'], 1, 1)
    x = _conv_block(x, params['conv5_w'], params['conv5_b'],
                    None, None, 1, 2, norm=False, act='tanh')
    return x


# speedup vs baseline: 2.8101x; 2.8101x over previous
import functools

import jax
import jax.numpy as jnp
from jax.experimental import pallas as pl
from jax.experimental.pallas import tpu as pltpu

_RATE = 0.5
_ENC = [(3, 8, 5, 1, 2), (8, 16, 3, 2, 1), (16, 32, 3, 2, 1), (32, 64, 3, 2, 1), (64, 64, 3, 2, 1)]


def _norm_act_body(x_ref, g_ref, be_ref, o_ref, *, C, cnt, act):
    X = x_ref[...]
    L = X.shape[1]
    M = (jax.lax.broadcasted_iota(jnp.int32, (L, C), 0) % C ==
         jax.lax.broadcasted_iota(jnp.int32, (L, C), 1)).astype(jnp.float32)
    MT = (jax.lax.broadcasted_iota(jnp.int32, (C, L), 1) % C ==
          jax.lax.broadcasted_iota(jnp.int32, (C, L), 0)).astype(jnp.float32)
    colsum = jnp.sum(X, axis=0, keepdims=True)
    m_c = jax.lax.dot(colsum, M) / cnt
    mvec = jax.lax.dot(m_c, MT)
    Xc = X - mvec
    ss = jnp.sum(Xc * Xc, axis=0, keepdims=True)
    v_c = jax.lax.dot(ss, M) / cnt
    scale_c = g_ref[...] / jnp.sqrt(v_c + 1e-5)
    svec = jax.lax.dot(scale_c, MT)
    bvec = jax.lax.dot(be_ref[...], MT)
    y = Xc * svec + bvec
    if act == 'lrelu':
        y = jnp.where(y >= 0, y, 0.2 * y)
    o_ref[...] = y


def _norm_act(x, g, be, act='lrelu'):
    n, h, w, c = x.shape
    x2 = x.reshape(n * h, w * c)
    body = functools.partial(_norm_act_body, C=c, cnt=float(n * h * w), act=act)
    out2 = pl.pallas_call(
        body,
        out_shape=jax.ShapeDtypeStruct((n * h, w * c), jnp.float32),
    )(x2, g.reshape(1, c), be.reshape(1, c))
    return out2.reshape(n, h, w, c)


def _conv_block(x, w, b, g, be, stride, pad, norm=True, act='lrelu',
                dn=('NHWC', 'OIHW', 'NHWC')):
    if pad > 0:
        axes = (1, 2) if dn[0] == 'NHWC' else (2, 3)
        cfg = [(0, 0)] * 4
        cfg[axes[0]] = (pad, pad)
        cfg[axes[1]] = (pad, pad)
        x = jnp.pad(x, cfg, mode='reflect')
    x = jax.lax.conv_general_dilated(x, w, (stride, stride), 'VALID',
                                     dimension_numbers=dn)
    if norm:
        return _norm_act(x, g, be, act)
    x = x + (b[None, :, None, None] if dn[2] == 'NCHW' else b[None, None, None, :])
    if act == 'lrelu':
        x = jnp.where(x >= 0, x, 0.2 * x)
    elif act == 'tanh':
        x = jnp.tanh(x)
    return x


def _up2(x):
    return jnp.repeat(jnp.repeat(x, 2, axis=1), 2, axis=2)


def _fusion_body(q_ref, fi_ref, sim_ref, out_ref, *, n, hw, num):
    b_idx = pl.program_id(0)
    featT = q_ref[0, 0]
    fi = fi_ref[0]

    fnorm = jnp.sqrt(jnp.sum(featT * featT, axis=1, keepdims=True))
    wfT = featT / jnp.maximum(fnorm, 1e-12)

    hidx = jax.lax.broadcasted_iota(jnp.int32, (num, hw), 1)
    GT = (hidx == fi).astype(jnp.float32)

    feat_selT = jax.lax.dot(GT, featT)
    wfsT = jax.lax.dot(GT, wfT)
    wnorm = jnp.sqrt(jnp.sum(wfsT * wfsT, axis=1, keepdims=True))
    wfsT = wfsT / jnp.maximum(wnorm, 1e-12)

    acc = sim_ref[b_idx, 0] * feat_selT
    hiota = jax.lax.broadcasted_iota(jnp.int32, (num, hw), 1)
    for j in range(n):
        refT = q_ref[0, 1 + j]
        rnorm = jnp.sqrt(jnp.sum(refT * refT, axis=1, keepdims=True))
        wrT = refT / jnp.maximum(rnorm, 1e-12)
        fx = jax.lax.dot_general(wfsT, wrT, (((1,), (1,)), ((), ())))
        maxv = jnp.max(fx, axis=1, keepdims=True)
        eligible = fx >= maxv
        ind = jnp.min(jnp.where(eligible, hiota, hw), axis=1, keepdims=True)
        onehot = (hiota == ind).astype(jnp.float32)
        ref_selT = jax.lax.dot(onehot, refT)
        acc = acc + sim_ref[b_idx, 1 + j] * ref_selT

    covered = jax.lax.dot_general(GT, jnp.ones((num, 1), jnp.float32),
                                  (((0,), (0,)), ((), ())))
    scattered = jax.lax.dot_general(GT, acc, (((0,), (0,)), ((), ())))
    out_ref[0] = featT * (1.0 - covered) + scattered


def _fusion_pallas(q, fi, sim):
    b, k, hw, c = q.shape
    n = k - 1
    num = fi.shape[1]
    body = functools.partial(_fusion_body, n=n, hw=hw, num=num)
    return pl.pallas_call(
        body,
        grid=(b,),
        in_specs=[
            pl.BlockSpec((1, k, hw, c), lambda i: (i, 0, 0, 0)),
            pl.BlockSpec((1, num, 1), lambda i: (i, 0, 0)),
            pl.BlockSpec(memory_space=pltpu.SMEM),
        ],
        out_specs=pl.BlockSpec((1, hw, c), lambda i: (i, 0, 0)),
        out_shape=jax.ShapeDtypeStruct((b, hw, c), jnp.float32),
    )(q, fi, sim)


def kernel(xs, params):
    b, k, cc, hh, ww = xs.shape
    x = xs.reshape(b * k, cc, hh, ww)
    feats = []
    for li, (ci, co, kk, st, pd) in enumerate(_ENC):
        dn = ('NCHW', 'OIHW', 'NHWC') if li == 0 else ('NHWC', 'OIHW', 'NHWC')
        x = _conv_block(x, params['enc%d_w' % li], params['enc%d_b' % li],
                        params['enc%d_g' % li], params['enc%d_be' % li], st, pd,
                        dn=dn)
        feats.append(x)
    x5 = feats[-1]
    h, w, c = x5.shape[1], x5.shape[2], x5.shape[3]
    hw = h * w
    num = int(_RATE * hw)
    sim = jax.random.uniform(jax.random.key(42), (b, k), jnp.float32)
    sim = sim / jnp.sum(sim, axis=1, keepdims=True)

    idx_keys = jax.random.split(jax.random.key(7), b)
    feat_indices = jnp.stack(
        [jax.random.permutation(idx_keys[i], hw)[:num] for i in range(b)])
    fi = feat_indices.astype(jnp.int32).reshape(b, num, 1)

    q = x5.reshape(b, k, hw, c)
    feat_gen = _fusion_pallas(q, fi, sim).reshape(b, h, w, c)

    skips = [f.reshape(b, k, f.shape[1], f.shape[2], f.shape[3])[:, 0]
             for f in feats[:-1]]
    x = _up2(feat_gen)
    s4 = _conv_block(skips[3], params['skip1_w'], params['skip1_b'],
                     params['skip1_g'], params['skip1_be'], 1, 0)
    x = jnp.concatenate([x, s4], axis=3)
    x = _conv_block(x, params['conv1_w'], params['conv1_b'],
                    params['conv1_g'], params['conv1_be'], 1, 1)
    x = _up2(x)
    s3 = _conv_block(skips[2], params['skip2_w'], params['skip2_b'],
                     params['skip2_g'], params['skip2_be'], 1, 0)
    x = jnp.concatenate([x, s3], axis=3)
    x = _conv_block(x, params['conv2_w'], params['conv2_b'],
                    params['conv2_g'], params['conv2_be'], 1, 1)
    x = _up2(x)
    x = _conv_block(x, params['conv3_w'], params['conv3_b'],
                    params['conv3_g'], params['conv3_be'], 1, 1)
    x = _up2(x)
    x = _conv_block(x, params['conv4_w'], params['conv4_b'],
                    params['conv4_g'], params['conv4_be'], 1, 1)
    x = _conv_block(x, params['conv5_w'], params['conv5_b'],
                    None, None, 1, 2, norm=False, act='tanh',
                    dn=('NHWC', 'OIHW', 'NCHW'))
    return x


# speedup vs baseline: 4.1752x; 1.4858x over previous
import functools

import jax
import jax.numpy as jnp
from jax.experimental import pallas as pl
from jax.experimental.pallas import tpu as pltpu

_RATE = 0.5
_ENC = [(3, 8, 5, 1, 2), (8, 16, 3, 2, 1), (16, 32, 3, 2, 1), (32, 64, 3, 2, 1), (64, 64, 3, 2, 1)]


def _conv_block_ref(x, w, b, g, be, stride, pad, norm=True, act='lrelu'):
    if pad > 0:
        x = jnp.pad(x, ((0, 0), (0, 0), (pad, pad), (pad, pad)), mode='reflect')
    x = jax.lax.conv_general_dilated(x, w, (stride, stride), 'VALID',
                                     dimension_numbers=('NCHW', 'OIHW', 'NCHW'))
    x = x + b[None, :, None, None]
    if norm:
        m = x.mean(axis=(0, 2, 3), keepdims=True)
        v = x.var(axis=(0, 2, 3), keepdims=True)
        x = (x - m) / jnp.sqrt(v + 1e-5)
        x = x * g[None, :, None, None] + be[None, :, None, None]
    if act == 'lrelu':
        x = jnp.where(x >= 0, x, 0.2 * x)
    elif act == 'tanh':
        x = jnp.tanh(x)
    return x


def _norm_act_body(x_ref, g_ref, be_ref, o_ref, *, C, cnt, act):
    X = x_ref[...]
    R = X.shape[0]
    MT = (jax.lax.broadcasted_iota(jnp.int32, (C, R), 1) % C ==
          jax.lax.broadcasted_iota(jnp.int32, (C, R), 0)).astype(jnp.float32)
    rowsum = jnp.sum(X, axis=1, keepdims=True)
    m_c = jax.lax.dot(MT, rowsum) / cnt
    mvec = jax.lax.dot_general(MT, m_c, (((0,), (0,)), ((), ())))
    Xc = X - mvec
    ssq = jnp.sum(Xc * Xc, axis=1, keepdims=True)
    v_c = jax.lax.dot(MT, ssq) / cnt
    scale_c = g_ref[...] / jnp.sqrt(v_c + 1e-5)
    svec = jax.lax.dot_general(MT, scale_c, (((0,), (0,)), ((), ())))
    bvec = jax.lax.dot_general(MT, be_ref[...], (((0,), (0,)), ((), ())))
    y = Xc * svec + bvec
    if act == 'lrelu':
        y = jnp.where(y >= 0, y, 0.2 * y)
    o_ref[...] = y


def _norm_act(x, g, be, act='lrelu'):
    n, c, h, w = x.shape
    x2 = x.reshape(n * c, h * w)
    body = functools.partial(_norm_act_body, C=c, cnt=float(n * h * w), act=act)
    out2 = pl.pallas_call(
        body,
        out_shape=jax.ShapeDtypeStruct((n * c, h * w), jnp.float32),
    )(x2, g.reshape(c, 1), be.reshape(c, 1))
    return out2.reshape(n, c, h, w)


def _conv_block_fast(x, w, g, be, stride, pad, act='lrelu'):
    if pad > 0:
        x = jnp.pad(x, ((0, 0), (0, 0), (pad, pad), (pad, pad)), mode='reflect')
    x = jax.lax.conv_general_dilated(x, w, (stride, stride), 'VALID',
                                     dimension_numbers=('NCHW', 'OIHW', 'NCHW'))
    return _norm_act(x, g, be, act)


def _up2(x):
    return jnp.repeat(jnp.repeat(x, 2, axis=2), 2, axis=3)


def _fusion_body(q_ref, fi_ref, sim_ref, out_ref, *, n, hw, num):
    b_idx = pl.program_id(0)
    featf = q_ref[0, 0]
    fi = fi_ref[0]

    fnorm = jnp.sqrt(jnp.sum(featf * featf, axis=0, keepdims=True))
    wf = featf / jnp.maximum(fnorm, 1e-12)

    hidx = jax.lax.broadcasted_iota(jnp.int32, (hw, num), 0)
    G = (hidx == fi).astype(jnp.float32)

    feat_sel = jax.lax.dot(featf, G)
    wfs = jax.lax.dot(wf, G)
    wnorm = jnp.sqrt(jnp.sum(wfs * wfs, axis=0, keepdims=True))
    wfs = wfs / jnp.maximum(wnorm, 1e-12)

    racc = None
    miota = jax.lax.broadcasted_iota(jnp.int32, (num, hw), 1)
    for j in range(n):
        ref_j = q_ref[0, 1 + j]
        rnorm = jnp.sqrt(jnp.sum(ref_j * ref_j, axis=0, keepdims=True))
        wr = ref_j / jnp.maximum(rnorm, 1e-12)
        fx = jax.lax.dot_general(wfs, wr, (((0,), (0,)), ((), ())))
        maxv = jnp.max(fx, axis=1, keepdims=True)
        eligible = fx >= maxv
        ind = jnp.min(jnp.where(eligible, miota, hw), axis=1, keepdims=True)
        onehot = (miota == ind).astype(jnp.float32)
        ref_sel = jax.lax.dot_general(ref_j, onehot, (((1,), (1,)), ((), ())))
        term = sim_ref[b_idx, 1 + j] * ref_sel
        racc = term if racc is None else racc + term

    acc = sim_ref[b_idx, 0] * feat_sel + racc

    covered = jax.lax.dot_general(jnp.ones((1, num), jnp.float32), G,
                                  (((1,), (1,)), ((), ())))
    scattered = jax.lax.dot_general(acc, G, (((1,), (1,)), ((), ())))
    out_ref[0] = featf * (1.0 - covered) + scattered


def _fusion_pallas(q, fi, sim):
    b, k, c, hw = q.shape
    n = k - 1
    num = fi.shape[2]
    body = functools.partial(_fusion_body, n=n, hw=hw, num=num)
    return pl.pallas_call(
        body,
        grid=(b,),
        in_specs=[
            pl.BlockSpec((1, k, c, hw), lambda i: (i, 0, 0, 0)),
            pl.BlockSpec((1, 1, num), lambda i: (i, 0, 0)),
            pl.BlockSpec(memory_space=pltpu.SMEM),
        ],
        out_specs=pl.BlockSpec((1, c, hw), lambda i: (i, 0, 0)),
        out_shape=jax.ShapeDtypeStruct((b, c, hw), jnp.float32),
    )(q, fi, sim)


def kernel(xs, params):
    b, k, cc, hh, ww = xs.shape
    x = xs.reshape(b * k, cc, hh, ww)
    feats = []
    for li, (ci, co, kk, st, pd) in enumerate(_ENC):
        x = _conv_block_ref(x, params['enc%d_w' % li], params['enc%d_b' % li],
                            params['enc%d_g' % li], params['enc%d_be' % li], st, pd)
        feats.append(x)
    x5 = feats[-1]
    c, h, w = x5.shape[1], x5.shape[2], x5.shape[3]
    hw = h * w
    num = int(_RATE * hw)
    sim = jax.random.uniform(jax.random.key(42), (b, k), jnp.float32)
    sim = sim / jnp.sum(sim, axis=1, keepdims=True)

    idx_keys = jax.random.split(jax.random.key(7), b)
    feat_indices = jnp.stack(
        [jax.random.permutation(idx_keys[i], hw)[:num] for i in range(b)])
    fi = feat_indices.astype(jnp.int32).reshape(b, 1, num)

    q = x5.reshape(b, k, c, hw)
    feat_gen = _fusion_pallas(q, fi, sim).reshape(b, c, h, w)

    skips = [f.reshape(b, k, f.shape[1], f.shape[2], f.shape[3])[:, 0]
             for f in feats[:-1]]
    x = _up2(feat_gen)
    s4 = _conv_block_fast(skips[3], params['skip1_w'],
                          params['skip1_g'], params['skip1_be'], 1, 0)
    x = jnp.concatenate([x, s4], axis=1)
    x = _conv_block_fast(x, params['conv1_w'],
                         params['conv1_g'], params['conv1_be'], 1, 1)
    x = _up2(x)
    s3 = _conv_block_fast(skips[2], params['skip2_w'],
                          params['skip2_g'], params['skip2_be'], 1, 0)
    x = jnp.concatenate([x, s3], axis=1)
    x = _conv_block_fast(x, params['conv2_w'],
                         params['conv2_g'], params['conv2_be'], 1, 1)
    x = _up2(x)
    x = _conv_block_fast(x, params['conv3_w'],
                         params['conv3_g'], params['conv3_be'], 1, 1)
    x = _up2(x)
    x = _conv_block_fast(x, params['conv4_w'],
                         params['conv4_g'], params['conv4_be'], 1, 1)
    x = _conv_block_ref(x, params['conv5_w'], params['conv5_b'],
                        None, None, 1, 2, norm=False, act='tanh')
    return x


# speedup vs baseline: 4.9439x; 1.1841x over previous
import functools

import jax
import jax.numpy as jnp
from jax.experimental import pallas as pl
from jax.experimental.pallas import tpu as pltpu

_RATE = 0.5
_ENC = [(3, 8, 5, 1, 2), (8, 16, 3, 2, 1), (16, 32, 3, 2, 1), (32, 64, 3, 2, 1), (64, 64, 3, 2, 1)]


def _conv_block(x, w, b, g, be, stride, pad, norm=True, act='lrelu'):
    if pad > 0:
        x = jnp.pad(x, ((0, 0), (0, 0), (pad, pad), (pad, pad)), mode='reflect')
    x = jax.lax.conv_general_dilated(x, w, (stride, stride), 'VALID',
                                     dimension_numbers=('NCHW', 'OIHW', 'NCHW'))
    x = x + b[None, :, None, None]
    if norm:
        m = x.mean(axis=(0, 2, 3), keepdims=True)
        v = x.var(axis=(0, 2, 3), keepdims=True)
        x = (x - m) / jnp.sqrt(v + 1e-5)
        x = x * g[None, :, None, None] + be[None, :, None, None]
    if act == 'lrelu':
        x = jnp.where(x >= 0, x, 0.2 * x)
    elif act == 'tanh':
        x = jnp.tanh(x)
    return x


def _up2(x):
    return jnp.repeat(jnp.repeat(x, 2, axis=2), 2, axis=3)


def _fusion_body(q_ref, fi_ref, sim_ref, out_ref, *, n, hw, num):
    b_idx = pl.program_id(0)
    featT = jnp.transpose(q_ref[0, 0], (1, 0))
    fi = fi_ref[0]

    fnorm = jnp.sqrt(jnp.sum(featT * featT, axis=1, keepdims=True))
    wfT = featT / jnp.maximum(fnorm, 1e-12)

    hidx = jax.lax.broadcasted_iota(jnp.int32, (num, hw), 1)
    GT = (hidx == fi).astype(jnp.float32)

    feat_selT = jax.lax.dot(GT, featT)
    wfsT = jax.lax.dot(GT, wfT)
    wnorm = jnp.sqrt(jnp.sum(wfsT * wfsT, axis=1, keepdims=True))
    wfsT = wfsT / jnp.maximum(wnorm, 1e-12)

    base_sim = sim_ref[b_idx, 0]
    acc = base_sim * feat_selT
    hiota = jax.lax.broadcasted_iota(jnp.int32, (num, hw), 1)
    for j in range(n):
        refT = jnp.transpose(q_ref[0, 1 + j], (1, 0))
        rnorm = jnp.sqrt(jnp.sum(refT * refT, axis=1, keepdims=True))
        wrT = refT / jnp.maximum(rnorm, 1e-12)
        fx = jax.lax.dot_general(wfsT, wrT, (((1,), (1,)), ((), ())))
        maxv = jnp.max(fx, axis=1, keepdims=True)
        eligible = fx >= maxv
        ind = jnp.min(jnp.where(eligible, hiota, hw), axis=1, keepdims=True)
        onehot = (hiota == ind).astype(jnp.float32)
        ref_selT = jax.lax.dot(onehot, refT)
        acc = acc + sim_ref[b_idx, 1 + j] * ref_selT

    covered = jax.lax.dot_general(GT, jnp.ones((num, 1), jnp.float32),
                                  (((0,), (0,)), ((), ())))
    scattered = jax.lax.dot_general(GT, acc, (((0,), (0,)), ((), ())))
    out_ref[0] = jnp.transpose(featT * (1.0 - covered) + scattered, (1, 0))


def _fusion_pallas(q, fi, sim):
    b, k, c, hw = q.shape
    n = k - 1
    num = fi.shape[1]
    body = functools.partial(_fusion_body, n=n, hw=hw, num=num)
    return pl.pallas_call(
        body,
        grid=(b,),
        in_specs=[
            pl.BlockSpec((1, k, c, hw), lambda i: (i, 0, 0, 0)),
            pl.BlockSpec((1, num, 1), lambda i: (i, 0, 0)),
            pl.BlockSpec(memory_space=pltpu.SMEM),
        ],
        out_specs=pl.BlockSpec((1, c, hw), lambda i: (i, 0, 0)),
        out_shape=jax.ShapeDtypeStruct((b, c, hw), jnp.float32),
    )(q, fi, sim)


def kernel(xs, params):
    b, k, cc, hh, ww = xs.shape
    x = xs.reshape(b * k, cc, hh, ww)
    feats = []
    for li, (ci, co, kk, st, pd) in enumerate(_ENC):
        x = _conv_block(x, params['enc%d_w' % li], params['enc%d_b' % li],
                        params['enc%d_g' % li], params['enc%d_be' % li], st, pd)
        feats.append(x)
    x5 = feats[-1]
    c, h, w = x5.shape[1], x5.shape[2], x5.shape[3]
    hw = h * w
    num = int(_RATE * hw)
    sim = jax.random.uniform(jax.random.key(42), (b, k), jnp.float32)
    sim = sim / jnp.sum(sim, axis=1, keepdims=True)

    idx_keys = jax.random.split(jax.random.key(7), b)
    feat_indices = jnp.stack(
        [jax.random.permutation(idx_keys[i], hw)[:num] for i in range(b)])
    fi = feat_indices.astype(jnp.int32).reshape(b, num, 1)

    q = x5.reshape(b, k, c, hw)
    feat_gen = _fusion_pallas(q, fi, sim).reshape(b, c, h, w)

    skips = [f.reshape(b, k, f.shape[1], f.shape[2], f.shape[3])[:, 0]
             for f in feats[:-1]]
    x = _up2(feat_gen)
    s4 = _conv_block(skips[3], params['skip1_w'], params['skip1_b'],
                     params['skip1_g'], params['skip1_be'], 1, 0)
    x = jnp.concatenate([x, s4], axis=1)
    x = _conv_block(x, params['conv1_w'], params['conv1_b'],
                    params['conv1_g'], params['conv1_be'], 1, 1)
    x = _up2(x)
    s3 = _conv_block(skips[2], params['skip2_w'], params['skip2_b'],
                     params['skip2_g'], params['skip2_be'], 1, 0)
    x = jnp.concatenate([x, s3], axis=1)
    x = _conv_block(x, params['conv2_w'], params['conv2_b'],
                    params['conv2_g'], params['conv2_be'], 1, 1)
    x = _up2(x)
    x = _conv_block(x, params['conv3_w'], params['conv3_b'],
                    params['conv3_g'], params['conv3_be'], 1, 1)
    x = _up2(x)
    x = _conv_block(x, params['conv4_w'], params['conv4_b'],
                    params['conv4_g'], params['conv4_be'], 1, 1)
    x = _conv_block(x, params['conv5_w'], params['conv5_b'],
                    None, None, 1, 2, norm=False, act='tanh')
    return x
